# Initial kernel scaffold; baseline (speedup 1.0000x reference)
import jax
import jax.numpy as jnp
from jax.experimental import pallas as pl


def kernel(x, W_in, b_in, Wq, bq, Wk, bk, Wv, bv, Wo, bo, ln1_g, ln1_b, W1, b1, W2, b2, ln2_g, ln2_b, W_out, b_out, Wg, bg, Wr, br, lno_g, lno_b):
    raise NotImplementedError("write your pallas kernel here")


# speedup vs baseline: 2.6923x; 2.6923x over previous
import functools

import jax
import jax.numpy as jnp
from jax.experimental import pallas as pl
import jax.experimental.pallas.tpu as pltpu

E = 8; K = 2; L = 2; H = 8; D = 64; S = 128; M = 512; F = 2048; O = 256; B = 64
DH = M // H
NJ = B * K


def _ln(h, g, b):
    mu = jnp.mean(h, -1, keepdims=True)
    v = jnp.mean((h - mu) ** 2, -1, keepdims=True)
    return (h - mu) / jnp.sqrt(v + 1e-5) * g + b


def _dot(a, b):
    return jnp.dot(a, b, preferred_element_type=jnp.float32)


def _gate_kernel(xg_ref, Wg_ref, bg_ref, Wr_ref, br_ref, ti_ref, tw_ref, r_ref):
    xg = xg_ref[...]
    logits = _dot(xg, Wg_ref[...]) + bg_ref[...]
    idx = jax.lax.broadcasted_iota(jnp.int32, (B, E), 1)
    m1 = jnp.max(logits, -1, keepdims=True)
    i1 = jnp.min(jnp.where(logits == m1, idx, E), -1, keepdims=True)
    masked = jnp.where(idx == i1, -1e30, logits)
    m2 = jnp.max(masked, -1, keepdims=True)
    i2 = jnp.min(jnp.where(masked == m2, idx, E), -1, keepdims=True)
    t2 = 1.0 / (1.0 + jnp.exp(m1 - m2))
    t1 = 1.0 - t2
    two = jax.lax.broadcasted_iota(jnp.int32, (B, K), 1)
    ti_ref[...] = jnp.where(two == 0, i1, i2).astype(jnp.int32)
    tw_ref[...] = jnp.where(two == 0, t1, t2)
    r_ref[...] = _dot(xg, Wr_ref[...]) + br_ref[...]


def _gate_call(xg, Wg, bg, Wr, br):
    return pl.pallas_call(
        _gate_kernel,
        out_shape=(
            jax.ShapeDtypeStruct((B, K), jnp.int32),
            jax.ShapeDtypeStruct((B, K), jnp.float32),
            jax.ShapeDtypeStruct((B, O), jnp.float32),
        ),
    )(xg, Wg, bg, Wr, br)


def _moe_kernel(jb, je, jw,
                x_ref, Win_ref, bin_ref,
                Wq_ref, bq_ref, Wk_ref, bk_ref, Wv_ref, bv_ref, Wo_ref, bo_ref,
                l1g_ref, l1b_ref, W1_ref, b1_ref, W2_ref, b2_ref,
                l2g_ref, l2b_ref, Wout_ref, bout_ref,
                r_ref, lnog_ref, lnob_ref,
                out_ref):
    j = pl.program_id(0)

    @pl.when(j == 0)
    def _init():
        out_ref[...] = jnp.zeros_like(out_ref)

    x = x_ref[0]
    h = _dot(x, Win_ref[0]) + bin_ref[0]
    for l in range(L):
        q = _dot(h, Wq_ref[0, l]) + bq_ref[0, l]
        k = _dot(h, Wk_ref[0, l]) + bk_ref[0, l]
        v = _dot(h, Wv_ref[0, l]) + bv_ref[0, l]
        parts = []
        for hh in range(H):
            sl = slice(hh * DH, (hh + 1) * DH)
            s = jax.lax.dot_general(
                q[:, sl], k[:, sl], (((1,), (1,)), ((), ())),
                preferred_element_type=jnp.float32) * (DH ** -0.5)
            s = jax.nn.softmax(s, axis=-1)
            parts.append(_dot(s, v[:, sl]))
        attn = jnp.concatenate(parts, axis=1)
        attn = _dot(attn, Wo_ref[0, l]) + bo_ref[0, l]
        h = _ln(h + attn, l1g_ref[0, l], l1b_ref[0, l])
        ff = jnp.maximum(_dot(h, W1_ref[0, l]) + b1_ref[0, l], 0.0)
        ff = _dot(ff, W2_ref[0, l]) + b2_ref[0, l]
        h = _ln(h + ff, l2g_ref[0, l], l2b_ref[0, l])
    pooled = jnp.mean(h, axis=0, keepdims=True)
    w = jw[j]
    y = _dot(pooled * w, Wout_ref[0]) + w * bout_ref[0]
    b = jb[j]
    out_ref[pl.ds(b, 1), :] += y

    @pl.when(j == NJ - 1)
    def _finish():
        acc = out_ref[...] + 0.1 * r_ref[...]
        mu = jnp.mean(acc, -1, keepdims=True)
        var = jnp.mean((acc - mu) ** 2, -1, keepdims=True)
        out_ref[...] = (acc - mu) / jnp.sqrt(var + 1e-5) * lnog_ref[...] + lnob_ref[...]


def _moe_call(job_batch, job_expert, job_w, x, W_in, b_in3,
              Wq, bq, Wk, bk, Wv, bv, Wo, bo, ln1_g, ln1_b,
              W1, b1, W2, b2, ln2_g, ln2_b, W_out, b_out3, r, lnog2, lnob2):
    def by_batch(i, jb, je, jw):
        return (jb[i], 0, 0)

    def by_exp(*dims):
        def f(i, jb, je, jw):
            return (je[i],) + (0,) * dims[0]
        return f

    def const(*dims):
        def f(i, jb, je, jw):
            return (0,) * dims[0]
        return f

    grid_spec = pltpu.PrefetchScalarGridSpec(
        num_scalar_prefetch=3,
        grid=(NJ,),
        in_specs=[
            pl.BlockSpec((1, S, D), by_batch),
            pl.BlockSpec((1, D, M), by_exp(2)),
            pl.BlockSpec((1, 1, M), by_exp(2)),
            pl.BlockSpec((1, L, M, M), by_exp(3)),
            pl.BlockSpec((1, L, M), by_exp(2)),
            pl.BlockSpec((1, L, M, M), by_exp(3)),
            pl.BlockSpec((1, L, M), by_exp(2)),
            pl.BlockSpec((1, L, M, M), by_exp(3)),
            pl.BlockSpec((1, L, M), by_exp(2)),
            pl.BlockSpec((1, L, M, M), by_exp(3)),
            pl.BlockSpec((1, L, M), by_exp(2)),
            pl.BlockSpec((1, L, M), by_exp(2)),
            pl.BlockSpec((1, L, M), by_exp(2)),
            pl.BlockSpec((1, L, M, F), by_exp(3)),
            pl.BlockSpec((1, L, F), by_exp(2)),
            pl.BlockSpec((1, L, F, M), by_exp(3)),
            pl.BlockSpec((1, L, M), by_exp(2)),
            pl.BlockSpec((1, L, M), by_exp(2)),
            pl.BlockSpec((1, L, M), by_exp(2)),
            pl.BlockSpec((1, M, O), by_exp(2)),
            pl.BlockSpec((1, 1, O), by_exp(2)),
            pl.BlockSpec((B, O), const(2)),
            pl.BlockSpec((1, O), const(2)),
            pl.BlockSpec((1, O), const(2)),
        ],
        out_specs=pl.BlockSpec((B, O), const(2)),
    )
    return pl.pallas_call(
        _moe_kernel,
        grid_spec=grid_spec,
        out_shape=jax.ShapeDtypeStruct((B, O), jnp.float32),
    )(job_batch, job_expert, job_w, x, W_in, b_in3,
      Wq, bq, Wk, bk, Wv, bv, Wo, bo, ln1_g, ln1_b,
      W1, b1, W2, b2, ln2_g, ln2_b, W_out, b_out3, r, lnog2, lnob2)


def kernel(x, W_in, b_in, Wq, bq, Wk, bk, Wv, bv, Wo, bo, ln1_g, ln1_b,
           W1, b1, W2, b2, ln2_g, ln2_b, W_out, b_out, Wg, bg, Wr, br,
           lno_g, lno_b):
    xg = x.reshape(B, S * D)
    ti, tw, r = _gate_call(xg, Wg, bg.reshape(1, E), Wr, br.reshape(1, O))

    e_flat = ti.reshape(-1)
    f = jnp.arange(NJ, dtype=jnp.int32)
    onehot = (e_flat[:, None] == jnp.arange(E, dtype=jnp.int32)[None, :]).astype(jnp.int32)
    cs = jnp.cumsum(onehot, 0)
    rank = jnp.sum((cs - onehot) * onehot, 1)
    counts = cs[-1]
    gstart = jnp.concatenate([jnp.zeros(1, jnp.int32),
                              jnp.cumsum(counts)[:-1].astype(jnp.int32)])
    pos = gstart[e_flat] + rank
    job_batch = jnp.zeros(NJ, jnp.int32).at[pos].set(f // K)
    job_expert = jnp.zeros(NJ, jnp.int32).at[pos].set(e_flat)
    job_w = jnp.zeros(NJ, jnp.float32).at[pos].set(tw.reshape(-1))

    return _moe_call(job_batch, job_expert, job_w, x, W_in,
                     b_in.reshape(E, 1, M), Wq, bq, Wk, bk, Wv, bv, Wo, bo,
                     ln1_g, ln1_b, W1, b1, W2, b2, ln2_g, ln2_b,
                     W_out, b_out.reshape(E, 1, O), r,
                     lno_g.reshape(1, O), lno_b.reshape(1, O))
